# baseline (speedup 1.0000x reference)
import functools
import math

import jax
import jax.numpy as jnp
from jax import lax
from jax.experimental import pallas as pl
from jax.experimental.pallas import tpu as pltpu
from jax.experimental.pallas import tpu_sc as plsc

LANES = 16


@functools.lru_cache(maxsize=None)
def _make_sc_gather(n_rows: int, d: int, k: int):
    info = plsc.get_sparse_core_info()
    nc, ns = info.num_cores, info.num_subcores
    nw = nc * ns
    assert n_rows % (nw * k) == 0
    rows_per_w = n_rows // nw
    n_chunks = rows_per_w // k
    scale = math.sqrt(float(d))
    mesh = plsc.VectorSubcoreMesh(core_axis_name="c", subcore_axis_name="s")

    @functools.partial(
        pl.kernel,
        mesh=mesh,
        out_type=jax.ShapeDtypeStruct((n_rows, d), jnp.float32),
        scratch_types=[
            pltpu.VMEM((n_chunks, k), jnp.int32),
            pltpu.VMEM((k, d), jnp.float32),
            pltpu.SemaphoreType.DMA,
        ],
    )
    def gather_scale(idx_hbm, table_hbm, out_hbm, idx_v, rows_v, sem):
        wid = lax.axis_index("s") * nc + lax.axis_index("c")
        base = wid * rows_per_w
        pltpu.sync_copy(idx_hbm.at[wid], idx_v)

        def chunk(c, carry):
            pltpu.async_copy(table_hbm.at[idx_v.at[c]], rows_v, sem).wait()

            def scale_slot(j, carry2):
                for r in range(k):
                    rows_v[r, pl.ds(j * LANES, LANES)] = (
                        rows_v[r, pl.ds(j * LANES, LANES)] * scale
                    )
                return carry2

            lax.fori_loop(0, d // LANES, scale_slot, 0, unroll=False)

            pltpu.sync_copy(rows_v, out_hbm.at[pl.ds(base + c * k, k)])
            return carry

        lax.fori_loop(0, n_chunks, chunk, 0, unroll=False)

    return gather_scale


def kernel(sequence, table):
    b, s = sequence.shape
    vocab, d = table.shape
    n_rows = b * s
    k = 16
    info = plsc.get_sparse_core_info()
    nw = info.num_cores * info.num_subcores
    idx = sequence.reshape(nw, (n_rows // nw) // k, k).astype(jnp.int32)
    fn = _make_sc_gather(n_rows, d, k)
    out = fn(idx, table)
    return out.reshape(b, s, d)
